# baseline (speedup 1.0000x reference)
import jax
import jax.numpy as jnp
from jax import lax
from jax.experimental import pallas as pl
from jax.experimental.pallas import tpu as pltpu
from jax.experimental.pallas import tpu_sc as plsc

LAMB_ = 0.7
N_ = 100000
D_ = 256
CH_ = 80
NCHUNK_ = N_ // CH_
NW_ = 32
CPW_ = NCHUNK_ // NW_
REM_ = NCHUNK_ % NW_
MAXC_ = CPW_ + 1


def _mix_body(x_hbm, idx_hbm, out_hbm, idx_all, gath_v, lin_v,
              load_sem, out_sem, idx_sem):
    c_id = lax.axis_index("c")
    s_id = lax.axis_index("s")
    wid = s_id * 2 + c_id
    nch = jnp.where(wid < REM_, CPW_ + 1, CPW_)

    def chunk_off(u):
        return (wid + u * NW_) * CH_

    def idx_issue(t, carry):
        pltpu.async_copy(idx_hbm.at[pl.ds(chunk_off(t), CH_)],
                         idx_all.at[t], idx_sem)
        return carry
    lax.fori_loop(0, nch, idx_issue, 0, unroll=1)

    def idx_drain(t, carry):
        pltpu.make_async_copy(idx_hbm.at[pl.ds(0, CH_)],
                              idx_all.at[0], idx_sem).wait()
        return carry
    lax.fori_loop(0, nch, idx_drain, 0, unroll=1)

    def start_loads(u, b):
        off = chunk_off(u)
        pltpu.async_copy(x_hbm.at[idx_all.at[u]], gath_v.at[b], load_sem[b])
        pltpu.async_copy(x_hbm.at[pl.ds(off, CH_)], lin_v.at[b], load_sem[b])

    def wait_loads(b):
        pltpu.make_async_copy(x_hbm.at[pl.ds(0, CH_)], gath_v.at[b],
                              load_sem[b]).wait()
        pltpu.make_async_copy(x_hbm.at[pl.ds(0, CH_)], lin_v.at[b],
                              load_sem[b]).wait()

    def start_out(u, b):
        pltpu.async_copy(lin_v.at[b], out_hbm.at[pl.ds(chunk_off(u), CH_)],
                         out_sem[b])

    def wait_out(b):
        pltpu.make_async_copy(x_hbm.at[pl.ds(0, CH_)], lin_v.at[b],
                              out_sem[b]).wait()

    def compute(b):
        def row_body(r, carry2):
            for j in range(D_ // 16):
                sl = pl.ds(j * 16, 16)
                a = lin_v[b, r, sl]
                g = gath_v[b, r, sl]
                lin_v[b, r, sl] = a * LAMB_ + g * (1.0 - LAMB_)
            return carry2
        lax.fori_loop(0, CH_, row_body, 0, unroll=1)

    start_loads(0, 0)

    def outer(i, carry):
        for b in range(2):
            u = i * 2 + b

            @pl.when(u < nch)
            def _():
                bo = 1 - b

                @pl.when(u + 1 < nch)
                def _():
                    @pl.when(u >= 1)
                    def _():
                        wait_out(bo)
                    start_loads(u + 1, bo)

                wait_loads(b)
                compute(b)
                start_out(u, b)
        return carry

    lax.fori_loop(0, MAXC_ // 2, outer, 0, unroll=1)
    wait_out(0)
    wait_out(1)


@jax.jit
def _mix(x, pair_idx):
    mesh = plsc.VectorSubcoreMesh(
        core_axis_name="c", subcore_axis_name="s", num_cores=2, num_subcores=16
    )
    return pl.kernel(
        _mix_body,
        out_type=jax.ShapeDtypeStruct((N_, D_), jnp.float32),
        mesh=mesh,
        scratch_types=[
            pltpu.VMEM((MAXC_, CH_), jnp.int32),
            pltpu.VMEM((2, CH_, D_), jnp.float32),
            pltpu.VMEM((2, CH_, D_), jnp.float32),
            [pltpu.SemaphoreType.DMA, pltpu.SemaphoreType.DMA],
            [pltpu.SemaphoreType.DMA, pltpu.SemaphoreType.DMA],
            pltpu.SemaphoreType.DMA,
        ],
    )(x, pair_idx)


def kernel(x, y, pair_idx):
    x_mix = _mix(x, pair_idx)
    return x_mix, y


# speedup vs baseline: 1.0679x; 1.0679x over previous
import jax
import jax.numpy as jnp
from jax import lax
from jax.experimental import pallas as pl
from jax.experimental.pallas import tpu as pltpu
from jax.experimental.pallas import tpu_sc as plsc

LAMB_ = 0.7
N_ = 100000
D_ = 256
CH_ = 80
NCHUNK_ = N_ // CH_
NW_ = 32
CPW_ = NCHUNK_ // NW_
REM_ = NCHUNK_ % NW_
MAXC_ = CPW_ + 1


def _mix_body(x_hbm, idx_hbm, out_hbm, idx_all, gath_v, lin_v, mix_v,
              load_sem, out_sem, idx_sem):
    c_id = lax.axis_index("c")
    s_id = lax.axis_index("s")
    wid = s_id * 2 + c_id
    nch = jnp.where(wid < REM_, CPW_ + 1, CPW_)

    def chunk_off(u):
        return (wid + u * NW_) * CH_

    def idx_issue(t, carry):
        pltpu.async_copy(idx_hbm.at[pl.ds(chunk_off(t), CH_)],
                         idx_all.at[t], idx_sem)
        return carry
    lax.fori_loop(0, nch, idx_issue, 0, unroll=1)

    def idx_drain(t, carry):
        pltpu.make_async_copy(idx_hbm.at[pl.ds(0, CH_)],
                              idx_all.at[0], idx_sem).wait()
        return carry
    lax.fori_loop(0, nch, idx_drain, 0, unroll=1)

    def start_loads(u, b):
        off = chunk_off(u)
        pltpu.async_copy(x_hbm.at[idx_all.at[u]], gath_v.at[b], load_sem[b])
        pltpu.async_copy(x_hbm.at[pl.ds(off, CH_)], lin_v.at[b], load_sem[b])

    def wait_loads(b):
        pltpu.make_async_copy(x_hbm.at[pl.ds(0, CH_)], gath_v.at[b],
                              load_sem[b]).wait()
        pltpu.make_async_copy(x_hbm.at[pl.ds(0, CH_)], lin_v.at[b],
                              load_sem[b]).wait()

    def start_out(u, b):
        pltpu.async_copy(mix_v.at[b], out_hbm.at[pl.ds(chunk_off(u), CH_)],
                         out_sem[b])

    def wait_out(b):
        pltpu.make_async_copy(x_hbm.at[pl.ds(0, CH_)], mix_v.at[b],
                              out_sem[b]).wait()

    def compute(b):
        def row_body(r, carry2):
            for j in range(D_ // 16):
                sl = pl.ds(j * 16, 16)
                mix_v[b, r, sl] = (lin_v[b, r, sl] * LAMB_
                                   + gath_v[b, r, sl] * (1.0 - LAMB_))
            return carry2
        lax.fori_loop(0, CH_, row_body, 0, unroll=1)

    start_loads(0, 0)

    def outer(i, carry):
        for b in range(2):
            u = i * 2 + b
            bo = 1 - b

            @pl.when(u < nch)
            def _():
                @pl.when(u + 1 < nch)
                def _():
                    start_loads(u + 1, bo)

                wait_loads(b)

                @pl.when(u >= 2)
                def _():
                    wait_out(b)

                compute(b)
                start_out(u, b)
        return carry

    lax.fori_loop(0, MAXC_ // 2, outer, 0, unroll=1)
    wait_out(0)
    wait_out(1)


@jax.jit
def _mix(x, pair_idx):
    mesh = plsc.VectorSubcoreMesh(
        core_axis_name="c", subcore_axis_name="s", num_cores=2, num_subcores=16
    )
    return pl.kernel(
        _mix_body,
        out_type=jax.ShapeDtypeStruct((N_, D_), jnp.float32),
        mesh=mesh,
        scratch_types=[
            pltpu.VMEM((MAXC_, CH_), jnp.int32),
            pltpu.VMEM((2, CH_, D_), jnp.float32),
            pltpu.VMEM((2, CH_, D_), jnp.float32),
            pltpu.VMEM((2, CH_, D_), jnp.float32),
            [pltpu.SemaphoreType.DMA, pltpu.SemaphoreType.DMA],
            [pltpu.SemaphoreType.DMA, pltpu.SemaphoreType.DMA],
            pltpu.SemaphoreType.DMA,
        ],
    )(x, pair_idx)


def kernel(x, y, pair_idx):
    x_mix = _mix(x, pair_idx)
    return x_mix, y
